# baseline (speedup 1.0000x reference)
import functools

import jax
import jax.numpy as jnp
from jax import lax
from jax.experimental import pallas as pl
from jax.experimental.pallas import tpu as pltpu
from jax.experimental.pallas import tpu_sc as plsc

B = 4096
L = 200
N = B * L
WORD_DIM = 64
WORD_PAD = 128
FEAT_LEN = 512
FEAT_DIM = 16
OUT_DIM = 96

NC, NS = 2, 16
NW = NC * NS
SUB = 128
NCH = N // (NW * SUB)


def _sc_embed(x2, l2, r2, wv, p1, p2):
    mesh = plsc.VectorSubcoreMesh(core_axis_name="c", subcore_axis_name="s")

    @functools.partial(
        pl.kernel,
        mesh=mesh,
        out_type=jax.ShapeDtypeStruct((N, OUT_DIM), jnp.float32),
        scratch_types=[
            pltpu.VMEM((SUB,), jnp.int32),
            pltpu.VMEM((SUB,), jnp.int32),
            pltpu.VMEM((SUB,), jnp.int32),
            pltpu.VMEM((SUB,), jnp.int32),
            pltpu.VMEM((SUB,), jnp.int32),
            pltpu.VMEM((SUB,), jnp.int32),
            pltpu.VMEM((SUB, WORD_PAD), jnp.float32),
            pltpu.VMEM((SUB, WORD_PAD), jnp.float32),
            pltpu.VMEM((SUB, WORD_PAD), jnp.float32),
            pltpu.VMEM((SUB, WORD_PAD), jnp.float32),
            pltpu.VMEM((SUB, OUT_DIM), jnp.float32),
            pltpu.VMEM((SUB, OUT_DIM), jnp.float32),
            pltpu.VMEM_SHARED((FEAT_LEN, WORD_PAD), jnp.float32),
            pltpu.VMEM_SHARED((FEAT_LEN, WORD_PAD), jnp.float32),
            pltpu.SemaphoreType.DMA,
            pltpu.SemaphoreType.DMA,
            pltpu.SemaphoreType.DMA,
            pltpu.SemaphoreType.DMA,
            pltpu.SemaphoreType.DMA,
            pltpu.SemaphoreType.DMA,
            pltpu.SemaphoreType.DMA,
        ],
    )
    def k(xh, lh, rh, wvh, p1h, p2h, outh,
          xi0, xi1, li0, li1, ri0, ri1,
          wvb0, wvb1, p1b, p2b, rows0, rows1, p1s, p2s,
          si0, si1, sg0, sg1, sp, so0, so1):
        cid = lax.axis_index("c")
        sid = lax.axis_index("s")
        wid = sid * NC + cid
        row0 = wid * NCH

        xi = (xi0, xi1)
        li = (li0, li1)
        ri = (ri0, ri1)
        wvb = (wvb0, wvb1)
        rows = (rows0, rows1)
        s_idx = (si0, si1)
        s_gat = (sg0, sg1)
        s_out = (so0, so1)

        pltpu.sync_copy(p1h, p1s)
        pltpu.sync_copy(p2h, p2s)

        def idx_cp(c, slot):
            r = row0 + c
            s = s_idx[slot]
            return (pltpu.make_async_copy(xh.at[r], xi[slot], s),
                    pltpu.make_async_copy(lh.at[r], li[slot], s),
                    pltpu.make_async_copy(rh.at[r], ri[slot], s))

        def wv_cp(c, slot):
            return (pltpu.make_async_copy(wvh.at[xi[slot]], wvb[slot],
                                          s_gat[slot]),)

        def pf_cp(c, slot):
            return (pltpu.make_async_copy(p1s.at[li[slot]], p1b, sp),
                    pltpu.make_async_copy(p2s.at[ri[slot]], p2b, sp))

        def out_cp(c, slot):
            base = (row0 + c) * SUB
            return (pltpu.make_async_copy(
                rows[slot], outh.at[pl.ds(base, SUB)], s_out[slot]),)

        def start(cs):
            for cp in cs:
                cp.start()

        def wait(cs):
            for cp in cs:
                cp.wait()

        def assemble(slot):
            rb, wb, pb1, pb2 = rows[slot], wvb[slot], p1b, p2b

            def body(g, carry):
                for u in range(4):
                    j = g * 4 + u
                    for c0 in range(4):
                        rb[j, pl.ds(c0 * 16, 16)] = wb[j, pl.ds(c0 * 16, 16)]
                    rb[j, pl.ds(64, 16)] = pb1[j, pl.ds(0, 16)]
                    rb[j, pl.ds(80, 16)] = pb2[j, pl.ds(0, 16)]
                return carry

            lax.fori_loop(0, SUB // 4, body, 0)

        start(idx_cp(0, 0))
        start(idx_cp(1, 1))
        wait(idx_cp(0, 0))
        start(wv_cp(0, 0))
        start(pf_cp(0, 0))

        def do_chunk(c, slot):
            @pl.when(c >= 2)
            def _():
                wait(out_cp(c - 2, slot))

            wait(wv_cp(c, slot))
            wait(pf_cp(c, slot))

            @pl.when(c + 1 < NCH)
            def _():
                wait(idx_cp(c + 1, slot ^ 1))
                start(wv_cp(c + 1, slot ^ 1))

            @pl.when(c + 2 < NCH)
            def _():
                start(idx_cp(c + 2, slot))

            assemble(slot)
            start(out_cp(c, slot))

            @pl.when(c + 1 < NCH)
            def _():
                start(pf_cp(c + 1, slot ^ 1))

        def body(i, carry):
            do_chunk(2 * i, 0)
            do_chunk(2 * i + 1, 1)
            return carry

        lax.fori_loop(0, NCH // 2, body, 0)
        wait(out_cp(NCH - 2, 0))
        wait(out_cp(NCH - 1, 1))

    return k(x2, l2, r2, wv, p1, p2)


JT = OUT_DIM // 8
LB = 8


def _tc_relayout(out2):
    view3 = out2.reshape(B, L, OUT_DIM)

    def body(in_ref, out_ref):
        v = in_ref[...]
        t = jnp.transpose(v, (1, 2, 0))
        out_ref[...] = t.reshape(LB, JT, 1, 8, SUB)

    return pl.pallas_call(
        body,
        grid=(L // LB, NW),
        in_specs=[pl.BlockSpec((SUB, LB, OUT_DIM), lambda lb, bt: (bt, lb, 0))],
        out_specs=pl.BlockSpec((LB, JT, 1, 8, SUB),
                               lambda lb, bt: (lb, 0, bt, 0, 0)),
        out_shape=jax.ShapeDtypeStruct((L, JT, NW, 8, SUB), jnp.float32),
    )(view3)


def kernel(x, ldist, rdist, Wv, pf1, pf2):
    x2 = x.reshape(N // SUB, SUB).astype(jnp.int32)
    l2 = ldist.reshape(N // SUB, SUB).astype(jnp.int32)
    r2 = rdist.reshape(N // SUB, SUB).astype(jnp.int32)
    wv128 = jnp.pad(Wv, ((0, 0), (0, WORD_PAD - WORD_DIM)))
    p1128 = jnp.pad(pf1, ((0, 0), (0, WORD_PAD - FEAT_DIM)))
    p2128 = jnp.pad(pf2, ((0, 0), (0, WORD_PAD - FEAT_DIM)))
    out2 = _sc_embed(x2, l2, r2, wv128, p1128, p2128)
    out5 = _tc_relayout(out2)
    return out5.transpose(2, 4, 0, 1, 3).reshape(B, 1, L, OUT_DIM)


# speedup vs baseline: 4.2745x; 4.2745x over previous
import functools

import jax
import jax.numpy as jnp
from jax import lax
from jax.experimental import pallas as pl
from jax.experimental.pallas import tpu as pltpu
from jax.experimental.pallas import tpu_sc as plsc

B = 4096
L = 200
N = B * L
WORD_DIM = 64
WORD_PAD = 128
FEAT_LEN = 512
FEAT_DIM = 16
OUT_DIM = 96

NC, NS = 2, 16
NW = NC * NS
SUB = 128
NCH = N // (NW * SUB)


def _sc_embed(x2, l2, r2, wv, p1, p2):
    mesh = plsc.VectorSubcoreMesh(core_axis_name="c", subcore_axis_name="s")

    @functools.partial(
        pl.kernel,
        mesh=mesh,
        out_type=jax.ShapeDtypeStruct((N, OUT_DIM), jnp.float32),
        scratch_types=[
            pltpu.VMEM((SUB,), jnp.int32),
            pltpu.VMEM((SUB,), jnp.int32),
            pltpu.VMEM((SUB,), jnp.int32),
            pltpu.VMEM((SUB,), jnp.int32),
            pltpu.VMEM((SUB,), jnp.int32),
            pltpu.VMEM((SUB,), jnp.int32),
            pltpu.VMEM((SUB, WORD_PAD), jnp.float32),
            pltpu.VMEM((SUB, WORD_PAD), jnp.float32),
            pltpu.VMEM((SUB, WORD_PAD), jnp.float32),
            pltpu.VMEM((SUB, WORD_PAD), jnp.float32),
            pltpu.VMEM((SUB, OUT_DIM), jnp.float32),
            pltpu.VMEM((SUB, OUT_DIM), jnp.float32),
            pltpu.VMEM_SHARED((FEAT_LEN, WORD_PAD), jnp.float32),
            pltpu.VMEM_SHARED((FEAT_LEN, WORD_PAD), jnp.float32),
            pltpu.SemaphoreType.DMA,
            pltpu.SemaphoreType.DMA,
            pltpu.SemaphoreType.DMA,
            pltpu.SemaphoreType.DMA,
            pltpu.SemaphoreType.DMA,
            pltpu.SemaphoreType.DMA,
            pltpu.SemaphoreType.DMA,
        ],
    )
    def k(xh, lh, rh, wvh, p1h, p2h, outh,
          xi0, xi1, li0, li1, ri0, ri1,
          wvb0, wvb1, p1b, p2b, rows0, rows1, p1s, p2s,
          si0, si1, sg0, sg1, sp, so0, so1):
        cid = lax.axis_index("c")
        sid = lax.axis_index("s")
        wid = sid * NC + cid
        row0 = wid * NCH

        xi = (xi0, xi1)
        li = (li0, li1)
        ri = (ri0, ri1)
        wvb = (wvb0, wvb1)
        rows = (rows0, rows1)
        s_idx = (si0, si1)
        s_gat = (sg0, sg1)
        s_out = (so0, so1)

        pltpu.sync_copy(p1h, p1s)
        pltpu.sync_copy(p2h, p2s)

        def idx_cp(c, slot):
            r = row0 + c
            s = s_idx[slot]
            return (pltpu.make_async_copy(xh.at[r], xi[slot], s),
                    pltpu.make_async_copy(lh.at[r], li[slot], s),
                    pltpu.make_async_copy(rh.at[r], ri[slot], s))

        def wv_cp(c, slot):
            return (pltpu.make_async_copy(wvh.at[xi[slot]], wvb[slot],
                                          s_gat[slot]),)

        def pf_cp(c, slot):
            return (pltpu.make_async_copy(p1s.at[li[slot]], p1b, sp),
                    pltpu.make_async_copy(p2s.at[ri[slot]], p2b, sp))

        def out_cp(c, slot):
            base = (row0 + c) * SUB
            return (pltpu.make_async_copy(
                rows[slot], outh.at[pl.ds(base, SUB)], s_out[slot]),)

        def start(cs):
            for cp in cs:
                cp.start()

        def wait(cs):
            for cp in cs:
                cp.wait()

        def assemble(slot):
            rb, wb, pb1, pb2 = rows[slot], wvb[slot], p1b, p2b

            def body(g, carry):
                for u in range(8):
                    j = g * 8 + u
                    for c0 in range(4):
                        rb[j, pl.ds(c0 * 16, 16)] = wb[j, pl.ds(c0 * 16, 16)]
                    rb[j, pl.ds(64, 16)] = pb1[j, pl.ds(0, 16)]
                    rb[j, pl.ds(80, 16)] = pb2[j, pl.ds(0, 16)]
                return carry

            lax.fori_loop(0, SUB // 8, body, 0)

        start(idx_cp(0, 0))
        start(idx_cp(1, 1))
        wait(idx_cp(0, 0))
        start(wv_cp(0, 0))
        start(pf_cp(0, 0))

        def do_chunk(c, slot):
            @pl.when(c >= 2)
            def _():
                wait(out_cp(c - 2, slot))

            wait(wv_cp(c, slot))
            wait(pf_cp(c, slot))

            @pl.when(c + 1 < NCH)
            def _():
                wait(idx_cp(c + 1, slot ^ 1))
                start(wv_cp(c + 1, slot ^ 1))

            @pl.when(c + 2 < NCH)
            def _():
                start(idx_cp(c + 2, slot))

            assemble(slot)
            start(out_cp(c, slot))

            @pl.when(c + 1 < NCH)
            def _():
                start(pf_cp(c + 1, slot ^ 1))

        def body(i, carry):
            do_chunk(2 * i, 0)
            do_chunk(2 * i + 1, 1)
            return carry

        lax.fori_loop(0, NCH // 2, body, 0)
        wait(out_cp(NCH - 2, 0))
        wait(out_cp(NCH - 1, 1))

    return k(x2, l2, r2, wv, p1, p2)


def kernel(x, ldist, rdist, Wv, pf1, pf2):
    x2 = x.reshape(N // SUB, SUB).astype(jnp.int32)
    l2 = ldist.reshape(N // SUB, SUB).astype(jnp.int32)
    r2 = rdist.reshape(N // SUB, SUB).astype(jnp.int32)
    wv128 = jnp.pad(Wv, ((0, 0), (0, WORD_PAD - WORD_DIM)))
    p1128 = jnp.pad(pf1, ((0, 0), (0, WORD_PAD - FEAT_DIM)))
    p2128 = jnp.pad(pf2, ((0, 0), (0, WORD_PAD - FEAT_DIM)))
    out2 = _sc_embed(x2, l2, r2, wv128, p1128, p2128)
    return out2.reshape(B, 1, L, OUT_DIM)


# speedup vs baseline: 4.4233x; 1.0348x over previous
import functools

import jax
import jax.numpy as jnp
from jax import lax
from jax.experimental import pallas as pl
from jax.experimental.pallas import tpu as pltpu
from jax.experimental.pallas import tpu_sc as plsc

B = 4096
L = 200
N = B * L
WORD_DIM = 64
WORD_PAD = 128
FEAT_LEN = 512
FEAT_DIM = 16
OUT_DIM = 96

NC, NS = 2, 16
NW = NC * NS
SUB = 128
NCH = N // (NW * SUB)


def _sc_embed(x2, l2, r2, wv, p1, p2):
    mesh = plsc.VectorSubcoreMesh(core_axis_name="c", subcore_axis_name="s")

    @functools.partial(
        pl.kernel,
        mesh=mesh,
        out_type=jax.ShapeDtypeStruct((N, OUT_DIM), jnp.float32),
        scratch_types=[
            pltpu.VMEM((SUB,), jnp.int32),
            pltpu.VMEM((SUB,), jnp.int32),
            pltpu.VMEM((SUB,), jnp.int32),
            pltpu.VMEM((SUB,), jnp.int32),
            pltpu.VMEM((SUB,), jnp.int32),
            pltpu.VMEM((SUB,), jnp.int32),
            pltpu.VMEM((SUB, WORD_PAD), jnp.float32),
            pltpu.VMEM((SUB, WORD_PAD), jnp.float32),
            pltpu.VMEM((SUB, WORD_PAD), jnp.float32),
            pltpu.VMEM((SUB, WORD_PAD), jnp.float32),
            pltpu.VMEM((SUB, OUT_DIM), jnp.float32),
            pltpu.VMEM((SUB, OUT_DIM), jnp.float32),
            pltpu.VMEM_SHARED((FEAT_LEN, WORD_PAD), jnp.float32),
            pltpu.VMEM_SHARED((FEAT_LEN, WORD_PAD), jnp.float32),
            pltpu.SemaphoreType.DMA,
            pltpu.SemaphoreType.DMA,
            pltpu.SemaphoreType.DMA,
            pltpu.SemaphoreType.DMA,
            pltpu.SemaphoreType.DMA,
            pltpu.SemaphoreType.DMA,
            pltpu.SemaphoreType.DMA,
        ],
    )
    def k(xh, lh, rh, wvh, p1h, p2h, outh,
          xi0, xi1, li0, li1, ri0, ri1,
          wvb0, wvb1, p1b, p2b, rows0, rows1, p1s, p2s,
          si0, si1, sg0, sg1, sp, so0, so1):
        cid = lax.axis_index("c")
        sid = lax.axis_index("s")
        wid = sid * NC + cid
        row0 = wid * NCH

        xi = (xi0, xi1)
        li = (li0, li1)
        ri = (ri0, ri1)
        wvb = (wvb0, wvb1)
        rows = (rows0, rows1)
        s_idx = (si0, si1)
        s_gat = (sg0, sg1)
        s_out = (so0, so1)

        pltpu.sync_copy(p1h, p1s)
        pltpu.sync_copy(p2h, p2s)

        def idx_cp(c, slot):
            r = row0 + c
            s = s_idx[slot]
            return (pltpu.make_async_copy(xh.at[r], xi[slot], s),
                    pltpu.make_async_copy(lh.at[r], li[slot], s),
                    pltpu.make_async_copy(rh.at[r], ri[slot], s))

        def wv_cp(c, slot):
            return (pltpu.make_async_copy(wvh.at[xi[slot]], wvb[slot],
                                          s_gat[slot]),)

        def pf_cp(c, slot):
            return (pltpu.make_async_copy(p1s.at[li[slot]], p1b, sp),
                    pltpu.make_async_copy(p2s.at[ri[slot]], p2b, sp))

        def out_cp(c, slot):
            base = (row0 + c) * SUB
            return (pltpu.make_async_copy(
                rows[slot], outh.at[pl.ds(base, SUB)], s_out[slot]),)

        def start(cs):
            for cp in cs:
                cp.start()

        def wait(cs):
            for cp in cs:
                cp.wait()

        def assemble_pf(slot):
            rb = rows[slot]

            def body(g, carry):
                for u in range(8):
                    j = g * 8 + u
                    rb[j, pl.ds(64, 16)] = p1b[j, pl.ds(0, 16)]
                    rb[j, pl.ds(80, 16)] = p2b[j, pl.ds(0, 16)]
                return carry

            lax.fori_loop(0, SUB // 8, body, 0)

        def assemble_wv(slot):
            rb, wb = rows[slot], wvb[slot]

            def body(g, carry):
                for u in range(8):
                    j = g * 8 + u
                    for c0 in range(4):
                        rb[j, pl.ds(c0 * 16, 16)] = wb[j, pl.ds(c0 * 16, 16)]
                return carry

            lax.fori_loop(0, SUB // 8, body, 0)

        start(idx_cp(0, 0))
        start(idx_cp(1, 1))
        wait(idx_cp(0, 0))
        start(wv_cp(0, 0))
        start(pf_cp(0, 0))

        def do_chunk(c, slot):
            @pl.when(c >= 2)
            def _():
                wait(out_cp(c - 2, slot))

            wait(wv_cp(c, slot))
            wait(pf_cp(c, slot))

            @pl.when(c + 1 < NCH)
            def _():
                wait(idx_cp(c + 1, slot ^ 1))
                start(wv_cp(c + 1, slot ^ 1))

            @pl.when(c + 2 < NCH)
            def _():
                start(idx_cp(c + 2, slot))

            assemble_pf(slot)

            @pl.when(c + 1 < NCH)
            def _():
                start(pf_cp(c + 1, slot ^ 1))

            assemble_wv(slot)
            start(out_cp(c, slot))

        def body(i, carry):
            do_chunk(2 * i, 0)
            do_chunk(2 * i + 1, 1)
            return carry

        lax.fori_loop(0, NCH // 2, body, 0)
        wait(out_cp(NCH - 2, 0))
        wait(out_cp(NCH - 1, 1))

    return k(x2, l2, r2, wv, p1, p2)


def kernel(x, ldist, rdist, Wv, pf1, pf2):
    x2 = x.reshape(N // SUB, SUB).astype(jnp.int32)
    l2 = ldist.reshape(N // SUB, SUB).astype(jnp.int32)
    r2 = rdist.reshape(N // SUB, SUB).astype(jnp.int32)
    wv128 = jnp.pad(Wv, ((0, 0), (0, WORD_PAD - WORD_DIM)))
    p1128 = jnp.pad(pf1, ((0, 0), (0, WORD_PAD - FEAT_DIM)))
    p2128 = jnp.pad(pf2, ((0, 0), (0, WORD_PAD - FEAT_DIM)))
    out2 = _sc_embed(x2, l2, r2, wv128, p1128, p2128)
    return out2.reshape(B, 1, L, OUT_DIM)


# speedup vs baseline: 4.4452x; 1.0049x over previous
import functools

import jax
import jax.numpy as jnp
from jax import lax
from jax.experimental import pallas as pl
from jax.experimental.pallas import tpu as pltpu
from jax.experimental.pallas import tpu_sc as plsc

B = 4096
L = 200
N = B * L
WORD_DIM = 64
WORD_PAD = 128
FEAT_LEN = 512
FEAT_DIM = 16
OUT_DIM = 96

NC, NS = 2, 16
NW = NC * NS
SUB = 128
NCH = N // (NW * SUB)


def _sc_embed(x2, l2, r2, wv, p1, p2):
    mesh = plsc.VectorSubcoreMesh(core_axis_name="c", subcore_axis_name="s")

    @functools.partial(
        pl.kernel,
        mesh=mesh,
        out_type=jax.ShapeDtypeStruct((N, OUT_DIM), jnp.float32),
        scratch_types=[
            pltpu.VMEM((SUB,), jnp.int32),
            pltpu.VMEM((SUB,), jnp.int32),
            pltpu.VMEM((SUB,), jnp.int32),
            pltpu.VMEM((SUB,), jnp.int32),
            pltpu.VMEM((SUB,), jnp.int32),
            pltpu.VMEM((SUB,), jnp.int32),
            pltpu.VMEM((SUB, WORD_PAD), jnp.float32),
            pltpu.VMEM((SUB, WORD_PAD), jnp.float32),
            pltpu.VMEM((SUB, WORD_PAD), jnp.float32),
            pltpu.VMEM((SUB, WORD_PAD), jnp.float32),
            pltpu.VMEM((SUB, OUT_DIM), jnp.float32),
            pltpu.VMEM((SUB, OUT_DIM), jnp.float32),
            pltpu.VMEM_SHARED((FEAT_LEN, WORD_PAD), jnp.float32),
            pltpu.VMEM_SHARED((FEAT_LEN, WORD_PAD), jnp.float32),
            pltpu.SemaphoreType.DMA,
            pltpu.SemaphoreType.DMA,
            pltpu.SemaphoreType.DMA,
            pltpu.SemaphoreType.DMA,
            pltpu.SemaphoreType.DMA,
            pltpu.SemaphoreType.DMA,
            pltpu.SemaphoreType.DMA,
        ],
    )
    def k(xh, lh, rh, wvh, p1h, p2h, outh,
          xi0, xi1, li0, li1, ri0, ri1,
          wvb0, wvb1, p1b, p2b, rows0, rows1, p1s, p2s,
          si0, si1, sg0, sg1, sp, so0, so1):
        cid = lax.axis_index("c")
        sid = lax.axis_index("s")
        wid = sid * NC + cid
        row0 = wid * NCH

        xi = (xi0, xi1)
        li = (li0, li1)
        ri = (ri0, ri1)
        wvb = (wvb0, wvb1)
        rows = (rows0, rows1)
        s_idx = (si0, si1)
        s_gat = (sg0, sg1)
        s_out = (so0, so1)

        pltpu.sync_copy(p1h, p1s)
        pltpu.sync_copy(p2h, p2s)

        def idx_cp(c, slot):
            r = row0 + c
            s = s_idx[slot]
            return (pltpu.make_async_copy(xh.at[r], xi[slot], s),
                    pltpu.make_async_copy(lh.at[r], li[slot], s),
                    pltpu.make_async_copy(rh.at[r], ri[slot], s))

        def wv_cp(c, slot):
            return (pltpu.make_async_copy(wvh.at[xi[slot]], wvb[slot],
                                          s_gat[slot]),)

        def pf_cp(c, slot):
            return (pltpu.make_async_copy(p1s.at[li[slot]], p1b, sp),
                    pltpu.make_async_copy(p2s.at[ri[slot]], p2b, sp))

        def out_cp(c, slot):
            base = (row0 + c) * SUB
            return (pltpu.make_async_copy(
                rows[slot], outh.at[pl.ds(base, SUB)], s_out[slot]),)

        def start(cs):
            for cp in cs:
                cp.start()

        def wait(cs):
            for cp in cs:
                cp.wait()

        def assemble_pf(slot):
            rb = rows[slot]

            def body(g, carry):
                for u in range(8):
                    j = g * 8 + u
                    rb[j, pl.ds(64, 16)] = p1b[j, pl.ds(0, 16)]
                    rb[j, pl.ds(80, 16)] = p2b[j, pl.ds(0, 16)]
                return carry

            lax.fori_loop(0, SUB // 8, body, 0)

        def assemble_wv(slot):
            rb, wb = rows[slot], wvb[slot]

            def body(g, carry):
                for u in range(8):
                    j = g * 8 + u
                    for c0 in range(4):
                        rb[j, pl.ds(c0 * 16, 16)] = wb[j, pl.ds(c0 * 16, 16)]
                return carry

            lax.fori_loop(0, SUB // 8, body, 0)

        start(idx_cp(0, 0))
        start(idx_cp(1, 1))
        wait(idx_cp(0, 0))
        start(wv_cp(0, 0))
        start(pf_cp(0, 0))

        def do_chunk(c, slot):
            @pl.when(c >= 2)
            def _():
                wait(out_cp(c - 2, slot))

            wait(wv_cp(c, slot))

            @pl.when(c + 1 < NCH)
            def _():
                wait(idx_cp(c + 1, slot ^ 1))
                start(wv_cp(c + 1, slot ^ 1))

            @pl.when(c + 2 < NCH)
            def _():
                start(idx_cp(c + 2, slot))

            wait(pf_cp(c, slot))

            assemble_pf(slot)

            @pl.when(c + 1 < NCH)
            def _():
                start(pf_cp(c + 1, slot ^ 1))

            assemble_wv(slot)
            start(out_cp(c, slot))

        def body(i, carry):
            do_chunk(2 * i, 0)
            do_chunk(2 * i + 1, 1)
            return carry

        lax.fori_loop(0, NCH // 2, body, 0)
        wait(out_cp(NCH - 2, 0))
        wait(out_cp(NCH - 1, 1))

    return k(x2, l2, r2, wv, p1, p2)


def kernel(x, ldist, rdist, Wv, pf1, pf2):
    x2 = x.reshape(N // SUB, SUB).astype(jnp.int32)
    l2 = ldist.reshape(N // SUB, SUB).astype(jnp.int32)
    r2 = rdist.reshape(N // SUB, SUB).astype(jnp.int32)
    wv128 = jnp.pad(Wv, ((0, 0), (0, WORD_PAD - WORD_DIM)))
    p1128 = jnp.pad(pf1, ((0, 0), (0, WORD_PAD - FEAT_DIM)))
    p2128 = jnp.pad(pf2, ((0, 0), (0, WORD_PAD - FEAT_DIM)))
    out2 = _sc_embed(x2, l2, r2, wv128, p1128, p2128)
    return out2.reshape(B, 1, L, OUT_DIM)
